# Initial kernel scaffold; baseline (speedup 1.0000x reference)
import jax
import jax.numpy as jnp
from jax.experimental import pallas as pl


def kernel(x, edge_index, edge_attr, u, batch, Wm1, bm1, Wm2, bm2, Wm3, bm3, Wn1, bn1, Wn2, bn2, Wn3, bn3):
    raise NotImplementedError("write your pallas kernel here")


# speedup vs baseline: 3.1975x; 3.1975x over previous
import functools

import jax
import jax.numpy as jnp
from jax import lax
from jax.experimental import pallas as pl
from jax.experimental.pallas import tpu as pltpu
from jax.experimental.pallas import tpu_sc as plsc

N = 10000
E = 320000
F = 128
H = 128

NC = 2
NS = 16
NW = NC * NS
GSZ = 128
NGRP = E // GSZ
NIT = (NGRP + NW - 1) // NW
RPT = 632
NPAD = NS * RPT

@functools.cache
def _build_sc_kernels():
    mesh = plsc.VectorSubcoreMesh(core_axis_name="c", subcore_axis_name="s",
                                  num_cores=NC, num_subcores=NS)

    @functools.partial(
        pl.kernel,
        out_type=jax.ShapeDtypeStruct((E, H), jnp.float32),
        mesh=mesh,
        scratch_types=[
            pltpu.VMEM((GSZ,), jnp.int32),
            pltpu.VMEM((GSZ, H), jnp.float32),
            pltpu.SemaphoreType.DMA,
        ],
    )
    def sc_gather(table_hbm, idx_hbm, out_hbm, idx_v, rows_v, sem):
        c = lax.axis_index("c")
        s = lax.axis_index("s")
        w = c * NS + s

        def body(i, carry):
            g = w + i * NW

            @pl.when(g < NGRP)
            def _():
                pltpu.sync_copy(idx_hbm.at[pl.ds(g * GSZ, GSZ)], idx_v)
                pltpu.async_copy(table_hbm.at[idx_v], rows_v, sem).wait()
                pltpu.sync_copy(rows_v, out_hbm.at[pl.ds(g * GSZ, GSZ), :])

            return carry

        lax.fori_loop(0, NIT, body, 0)

    @functools.partial(
        pl.kernel,
        out_type=jax.ShapeDtypeStruct((NC, NPAD, H), jnp.float32),
        mesh=mesh,
        scratch_types=[
            pltpu.VMEM((GSZ,), jnp.int32),
            pltpu.VMEM((GSZ, H), jnp.float32),
            pltpu.VMEM_SHARED((NPAD, H), jnp.float32),
        ],
    )
    def sc_scatter(m_hbm, idx_hbm, zeros_hbm, out_hbm, idx_v, rows_v, acc):
        c = lax.axis_index("c")
        s = lax.axis_index("s")
        w = c * NS + s

        pltpu.sync_copy(zeros_hbm, acc.at[pl.ds(s * RPT, RPT), :])
        plsc.subcore_barrier()

        def body(i, carry):
            g = w + i * NW

            @pl.when(g < NGRP)
            def _():
                pltpu.sync_copy(idx_hbm.at[pl.ds(g * GSZ, GSZ)], idx_v)
                pltpu.sync_copy(m_hbm.at[pl.ds(g * GSZ, GSZ), :], rows_v)
                pltpu.sync_copy(rows_v, acc.at[idx_v], add=True)

            return carry

        lax.fori_loop(0, NIT, body, 0)
        plsc.subcore_barrier()
        pltpu.sync_copy(acc.at[pl.ds(s * RPT, RPT), :],
                        out_hbm.at[c, pl.ds(s * RPT, RPT), :])

    return sc_gather, sc_scatter


BN = 2000
BE = 3200


def _xw_body(x_ref, w_ref, b_ref, o_ref):
    o_ref[...] = jnp.dot(x_ref[...], w_ref[...],
                         preferred_element_type=jnp.float32) + b_ref[...]


def _edge_body(xg_ref, ea_ref, w1b_ref, w2_ref, b2_ref, w3_ref, b3_ref, o_ref):
    m1 = jnp.maximum(
        xg_ref[...] + jnp.dot(ea_ref[...], w1b_ref[...],
                              preferred_element_type=jnp.float32), 0.0)
    m2 = jnp.maximum(
        jnp.dot(m1, w2_ref[...], preferred_element_type=jnp.float32)
        + b2_ref[...], 0.0)
    o_ref[...] = (jnp.dot(m2, w3_ref[...], preferred_element_type=jnp.float32)
                  + b3_ref[...])


def _node_body(x_ref, p_ref, w1a_ref, w1b_ref, b1_ref, w2_ref, b2_ref,
               w3_ref, b3_ref, o_ref):
    agg = p_ref[0] + p_ref[1]
    h1 = jnp.maximum(
        jnp.dot(x_ref[...], w1a_ref[...], preferred_element_type=jnp.float32)
        + jnp.dot(agg, w1b_ref[...], preferred_element_type=jnp.float32)
        + b1_ref[...], 0.0)
    h2 = jnp.maximum(
        jnp.dot(h1, w2_ref[...], preferred_element_type=jnp.float32)
        + b2_ref[...], 0.0)
    o_ref[...] = (jnp.dot(h2, w3_ref[...], preferred_element_type=jnp.float32)
                  + b3_ref[...])


def _w_spec(r, c_):
    return pl.BlockSpec((r, c_), lambda i: (0, 0))


def kernel(x, edge_index, edge_attr, u, batch, Wm1, bm1, Wm2, bm2, Wm3, bm3,
           Wn1, bn1, Wn2, bn2, Wn3, bn3):
    send_idx = edge_index[0]
    rec_idx = edge_index[1]
    b1 = bm1.reshape(1, H)
    b2 = bm2.reshape(1, H)
    b3 = bm3.reshape(1, H)
    n1 = bn1.reshape(1, H)
    n2 = bn2.reshape(1, H)
    n3 = bn3.reshape(1, H)
    zeros = jnp.zeros((RPT, H), jnp.float32)

    xw = pl.pallas_call(
        _xw_body,
        grid=(N // BN,),
        in_specs=[pl.BlockSpec((BN, F), lambda i: (i, 0)),
                  _w_spec(F, H), _w_spec(1, H)],
        out_specs=pl.BlockSpec((BN, H), lambda i: (i, 0)),
        out_shape=jax.ShapeDtypeStruct((N, H), jnp.float32),
    )(x, Wm1[:F], b1)

    sc_gather, sc_scatter = _build_sc_kernels()
    xg = sc_gather(xw, send_idx)

    m3 = pl.pallas_call(
        _edge_body,
        grid=(E // BE,),
        in_specs=[pl.BlockSpec((BE, H), lambda i: (i, 0)),
                  pl.BlockSpec((BE, H), lambda i: (i, 0)),
                  _w_spec(H, H), _w_spec(H, H), _w_spec(1, H),
                  _w_spec(H, H), _w_spec(1, H)],
        out_specs=pl.BlockSpec((BE, H), lambda i: (i, 0)),
        out_shape=jax.ShapeDtypeStruct((E, H), jnp.float32),
    )(xg, edge_attr, Wm1[F:], Wm2, b2, Wm3, b3)

    partials = sc_scatter(m3, rec_idx, zeros)[:, :N, :]

    out = pl.pallas_call(
        _node_body,
        grid=(N // BN,),
        in_specs=[pl.BlockSpec((BN, F), lambda i: (i, 0)),
                  pl.BlockSpec((NC, BN, H), lambda i: (0, i, 0)),
                  _w_spec(F, H), _w_spec(H, H), _w_spec(1, H),
                  _w_spec(H, H), _w_spec(1, H), _w_spec(H, H), _w_spec(1, H)],
        out_specs=pl.BlockSpec((BN, H), lambda i: (i, 0)),
        out_shape=jax.ShapeDtypeStruct((N, H), jnp.float32),
    )(x, partials, Wn1[:F], Wn1[F:], n1, Wn2, n2, Wn3, n3)

    return out


# speedup vs baseline: 4.2856x; 1.3403x over previous
import functools

import jax
import jax.numpy as jnp
from jax import lax
from jax.experimental import pallas as pl
from jax.experimental.pallas import tpu as pltpu
from jax.experimental.pallas import tpu_sc as plsc

N = 10000
E = 320000
F = 128
H = 128

NC = 2
NS = 16
NW = NC * NS
GSZ = 128
NGRP = E // GSZ
NIT = (NGRP + NW - 1) // NW
RPT = 632
NPAD = NS * RPT

@functools.cache
def _build_sc_kernels():
    mesh = plsc.VectorSubcoreMesh(core_axis_name="c", subcore_axis_name="s",
                                  num_cores=NC, num_subcores=NS)

    @functools.partial(
        pl.kernel,
        out_type=jax.ShapeDtypeStruct((E, H), jnp.float32),
        mesh=mesh,
        scratch_types=[
            pltpu.VMEM((2, GSZ), jnp.int32),
            pltpu.VMEM((2, GSZ, H), jnp.float32),
            pltpu.SemaphoreType.DMA,
            pltpu.SemaphoreType.DMA,
            pltpu.SemaphoreType.DMA,
            pltpu.SemaphoreType.DMA,
            pltpu.SemaphoreType.DMA,
        ],
    )
    def sc_gather(table_hbm, idx_hbm, out_hbm, idx_v, rows_v,
                  isem0, isem1, gsem, wsem0, wsem1):
        c = lax.axis_index("c")
        s = lax.axis_index("s")
        w = c * NS + s
        isems = (isem0, isem1)
        wsems = (wsem0, wsem1)

        pltpu.async_copy(idx_hbm.at[pl.ds(w * GSZ, GSZ)], idx_v.at[0], isem0)

        def body(i, carry):
            b = lax.rem(i, 2)
            g = w + i * NW

            @pl.when(g < NGRP)
            def _():
                gn = g + NW

                @pl.when(gn < NGRP)
                def _():
                    for nb in (0, 1):
                        @pl.when(b == 1 - nb)
                        def _():
                            pltpu.async_copy(
                                idx_hbm.at[pl.ds(gn * GSZ, GSZ)],
                                idx_v.at[nb], isems[nb])

                for cb in (0, 1):
                    @pl.when(b == cb)
                    def _():
                        pltpu.make_async_copy(
                            idx_hbm.at[pl.ds(g * GSZ, GSZ)],
                            idx_v.at[cb], isems[cb]).wait()
                        @pl.when(i >= 2)
                        def _():
                            pltpu.make_async_copy(
                                rows_v.at[cb],
                                out_hbm.at[pl.ds(g * GSZ, GSZ), :],
                                wsems[cb]).wait()
                        pltpu.async_copy(table_hbm.at[idx_v.at[cb]],
                                         rows_v.at[cb], gsem).wait()
                        pltpu.async_copy(rows_v.at[cb],
                                         out_hbm.at[pl.ds(g * GSZ, GSZ), :],
                                         wsems[cb])

            return carry

        lax.fori_loop(0, NIT, body, 0)
        for cb in (0, 1):
            pltpu.make_async_copy(rows_v.at[cb],
                                  out_hbm.at[pl.ds(0, GSZ), :],
                                  wsems[cb]).wait()

    @functools.partial(
        pl.kernel,
        out_type=jax.ShapeDtypeStruct((NC, NPAD, H), jnp.float32),
        mesh=mesh,
        scratch_types=[
            pltpu.VMEM((2, GSZ), jnp.int32),
            pltpu.VMEM((2, GSZ, H), jnp.float32),
            pltpu.VMEM_SHARED((NPAD, H), jnp.float32),
            pltpu.SemaphoreType.DMA,
            pltpu.SemaphoreType.DMA,
            pltpu.SemaphoreType.DMA,
            pltpu.SemaphoreType.DMA,
        ],
    )
    def sc_scatter(m_hbm, idx_hbm, zeros_hbm, out_hbm, idx_v, rows_v, acc,
                   isem0, isem1, rsem0, rsem1):
        c = lax.axis_index("c")
        s = lax.axis_index("s")
        w = c * NS + s
        isems = (isem0, isem1)
        rsems = (rsem0, rsem1)

        pltpu.sync_copy(zeros_hbm, acc.at[pl.ds(s * RPT, RPT), :])

        pltpu.async_copy(idx_hbm.at[pl.ds(w * GSZ, GSZ)], idx_v.at[0], isem0)
        pltpu.async_copy(m_hbm.at[pl.ds(w * GSZ, GSZ), :], rows_v.at[0], rsem0)
        plsc.subcore_barrier()

        def body(i, carry):
            b = lax.rem(i, 2)
            g = w + i * NW

            @pl.when(g < NGRP)
            def _():
                gn = g + NW

                @pl.when(gn < NGRP)
                def _():
                    for nb in (0, 1):
                        @pl.when(b == 1 - nb)
                        def _():
                            pltpu.async_copy(
                                idx_hbm.at[pl.ds(gn * GSZ, GSZ)],
                                idx_v.at[nb], isems[nb])
                            pltpu.async_copy(
                                m_hbm.at[pl.ds(gn * GSZ, GSZ), :],
                                rows_v.at[nb], rsems[nb])

                for cb in (0, 1):
                    @pl.when(b == cb)
                    def _():
                        pltpu.make_async_copy(
                            idx_hbm.at[pl.ds(g * GSZ, GSZ)],
                            idx_v.at[cb], isems[cb]).wait()
                        pltpu.make_async_copy(
                            m_hbm.at[pl.ds(g * GSZ, GSZ), :],
                            rows_v.at[cb], rsems[cb]).wait()
                        pltpu.sync_copy(rows_v.at[cb], acc.at[idx_v.at[cb]],
                                        add=True)

            return carry

        lax.fori_loop(0, NIT, body, 0)
        plsc.subcore_barrier()
        pltpu.sync_copy(acc.at[pl.ds(s * RPT, RPT), :],
                        out_hbm.at[c, pl.ds(s * RPT, RPT), :])

    return sc_gather, sc_scatter


BN = 2000
BE = 3200


def _xw_body(x_ref, w_ref, b_ref, o_ref):
    o_ref[...] = jnp.dot(x_ref[...], w_ref[...],
                         preferred_element_type=jnp.float32) + b_ref[...]


def _edge_body(xg_ref, ea_ref, w1b_ref, w2_ref, b2_ref, w3_ref, b3_ref, o_ref):
    m1 = jnp.maximum(
        xg_ref[...] + jnp.dot(ea_ref[...], w1b_ref[...],
                              preferred_element_type=jnp.float32), 0.0)
    m2 = jnp.maximum(
        jnp.dot(m1, w2_ref[...], preferred_element_type=jnp.float32)
        + b2_ref[...], 0.0)
    o_ref[...] = (jnp.dot(m2, w3_ref[...], preferred_element_type=jnp.float32)
                  + b3_ref[...])


def _node_body(x_ref, p_ref, w1a_ref, w1b_ref, b1_ref, w2_ref, b2_ref,
               w3_ref, b3_ref, o_ref):
    agg = p_ref[0] + p_ref[1]
    h1 = jnp.maximum(
        jnp.dot(x_ref[...], w1a_ref[...], preferred_element_type=jnp.float32)
        + jnp.dot(agg, w1b_ref[...], preferred_element_type=jnp.float32)
        + b1_ref[...], 0.0)
    h2 = jnp.maximum(
        jnp.dot(h1, w2_ref[...], preferred_element_type=jnp.float32)
        + b2_ref[...], 0.0)
    o_ref[...] = (jnp.dot(h2, w3_ref[...], preferred_element_type=jnp.float32)
                  + b3_ref[...])


def _w_spec(r, c_):
    return pl.BlockSpec((r, c_), lambda i: (0, 0))


def kernel(x, edge_index, edge_attr, u, batch, Wm1, bm1, Wm2, bm2, Wm3, bm3,
           Wn1, bn1, Wn2, bn2, Wn3, bn3):
    send_idx = edge_index[0]
    rec_idx = edge_index[1]
    b1 = bm1.reshape(1, H)
    b2 = bm2.reshape(1, H)
    b3 = bm3.reshape(1, H)
    n1 = bn1.reshape(1, H)
    n2 = bn2.reshape(1, H)
    n3 = bn3.reshape(1, H)
    zeros = jnp.zeros((RPT, H), jnp.float32)

    xw = pl.pallas_call(
        _xw_body,
        grid=(N // BN,),
        in_specs=[pl.BlockSpec((BN, F), lambda i: (i, 0)),
                  _w_spec(F, H), _w_spec(1, H)],
        out_specs=pl.BlockSpec((BN, H), lambda i: (i, 0)),
        out_shape=jax.ShapeDtypeStruct((N, H), jnp.float32),
    )(x, Wm1[:F], b1)

    sc_gather, sc_scatter = _build_sc_kernels()
    xg = sc_gather(xw, send_idx)

    m3 = pl.pallas_call(
        _edge_body,
        grid=(E // BE,),
        in_specs=[pl.BlockSpec((BE, H), lambda i: (i, 0)),
                  pl.BlockSpec((BE, H), lambda i: (i, 0)),
                  _w_spec(H, H), _w_spec(H, H), _w_spec(1, H),
                  _w_spec(H, H), _w_spec(1, H)],
        out_specs=pl.BlockSpec((BE, H), lambda i: (i, 0)),
        out_shape=jax.ShapeDtypeStruct((E, H), jnp.float32),
    )(xg, edge_attr, Wm1[F:], Wm2, b2, Wm3, b3)

    partials = sc_scatter(m3, rec_idx, zeros)[:, :N, :]

    out = pl.pallas_call(
        _node_body,
        grid=(N // BN,),
        in_specs=[pl.BlockSpec((BN, F), lambda i: (i, 0)),
                  pl.BlockSpec((NC, BN, H), lambda i: (0, i, 0)),
                  _w_spec(F, H), _w_spec(H, H), _w_spec(1, H),
                  _w_spec(H, H), _w_spec(1, H), _w_spec(H, H), _w_spec(1, H)],
        out_specs=pl.BlockSpec((BN, H), lambda i: (i, 0)),
        out_shape=jax.ShapeDtypeStruct((N, H), jnp.float32),
    )(x, partials, Wn1[:F], Wn1[F:], n1, Wn2, n2, Wn3, n3)

    return out
